# Initial kernel scaffold; baseline (speedup 1.0000x reference)
import jax
import jax.numpy as jnp
from jax.experimental import pallas as pl


def kernel(x, conv1_w, conv1_b, conv2_w, conv2_b, fc1_w, fc1_b, fc2_w, fc2_b):
    raise NotImplementedError("write your pallas kernel here")


# speedup vs baseline: 8.1154x; 8.1154x over previous
import jax
import jax.numpy as jnp
from jax.experimental import pallas as pl
from jax.experimental.pallas import tpu as pltpu

BF = jnp.bfloat16
F32 = jnp.float32
PHASES = ((0, 0), (0, 1), (1, 0), (1, 1))


def _conv1_kernel(xps_ref, w_ref, b_ref, o_ref, col_ref):
    SH = 10
    for s in range(5):
        for p, (dh, dw) in enumerate(PHASES):
            for kh in range(3):
                for kw in range(3):
                    u, v = dh + kh, dw + kw
                    g = p * 27 + (kh * 3 + kw) * 3
                    h0 = s * SH + u // 2
                    w0 = v // 2
                    col_ref[g:g + 3] = xps_ref[0, u % 2, v % 2, :,
                                               h0:h0 + SH, w0:w0 + 32, :]
        d = jnp.dot(w_ref[...], col_ref[...].reshape(108, SH * 32 * 128),
                    preferred_element_type=F32)
        m = jnp.maximum(jnp.maximum(d[0:16], d[16:32]),
                        jnp.maximum(d[32:48], d[48:64]))
        y = jnp.maximum(m + b_ref[...], 0.0)
        o_ref[:, s * SH:(s + 1) * SH, 0, :, :] = (
            y.reshape(16, SH, 32, 128).astype(BF))


def _conv2_mlp_kernel(yps_ref, w2_ref, b2_ref, fc1_ref, b3_ref, w4_ref,
                      b4_ref, o_ref, col_ref, feat_ref):
    SH = 5
    for s in range(5):
        m = None
        for dh, dw in PHASES:
            for kh in range(3):
                for kw in range(3):
                    u, v = dh + kh, dw + kw
                    g = (kh * 3 + kw) * 16
                    h0 = s * SH + u // 2
                    w0 = v // 2
                    col_ref[g:g + 16] = yps_ref[u % 2, v % 2, :,
                                                h0:h0 + SH, w0:w0 + 32, :]
            d = jnp.dot(w2_ref[...], col_ref[...].reshape(144, SH * 32 * 128),
                        preferred_element_type=F32)
            m = d if m is None else jnp.maximum(m, d)
        f = jnp.maximum(m + b2_ref[...], 0.0)
        feat_ref[:, s * SH:(s + 1) * SH, :, :] = (
            f.reshape(8, SH, 32, 128).astype(BF))
    h = jnp.dot(fc1_ref[...], feat_ref[...].reshape(6400, 128),
                preferred_element_type=F32)
    h = jnp.maximum(h + b3_ref[...], 0.0)
    y = jnp.dot(w4_ref[...], h.astype(BF), preferred_element_type=F32)
    o_ref[...] = jnp.maximum(y + b4_ref[...], 0.0)


def _block_diag4(w):
    m, k = w.shape
    out = jnp.zeros((4 * m, 4 * k), w.dtype)
    for p in range(4):
        out = out.at[p * m:(p + 1) * m, p * k:(p + 1) * k].set(w)
    return out


def kernel(x, conv1_w, conv1_b, conv2_w, conv2_b, fc1_w, fc1_b, fc2_w, fc2_b):
    B = x.shape[0]
    G = B // 128

    xp = jnp.pad(x.astype(BF), ((0, 0), (0, 0), (1, 1), (1, 1)))
    xr = xp.reshape(B, 3, 51, 2, 51, 2).transpose(3, 5, 1, 2, 4, 0)
    xr = jnp.pad(xr, ((0, 0),) * 4 + ((0, 14), (0, 0)))
    xps2 = jnp.stack([xr[:, :, :, :, 0:40, :],
                      xr[:, :, :, :, 25:65, :]], axis=0)
    w1 = _block_diag4(conv1_w.transpose(0, 2, 3, 1).reshape(16, 27).astype(BF))
    b1 = conv1_b.reshape(16, 1).astype(F32)

    y1w = pl.pallas_call(
        _conv1_kernel,
        out_shape=jax.ShapeDtypeStruct((16, 50, 2, 32, B), BF),
        grid=(G, 2),
        in_specs=[
            pl.BlockSpec((1, 2, 2, 3, 51, 40, 128),
                         lambda i, j: (j, 0, 0, 0, 0, 0, i)),
            pl.BlockSpec((64, 108), lambda i, j: (0, 0)),
            pl.BlockSpec((16, 1), lambda i, j: (0, 0)),
        ],
        out_specs=pl.BlockSpec((16, 50, 1, 32, 128),
                               lambda i, j: (0, 0, j, 0, i)),
        scratch_shapes=[pltpu.VMEM((108, 10, 32, 128), BF)],
        compiler_params=pltpu.CompilerParams(
            dimension_semantics=("parallel", "arbitrary"),
            vmem_limit_bytes=56 * 1024 * 1024),
    )(xps2, w1, b1)

    y1 = jnp.concatenate([y1w[:, :, 0, :25, :], y1w[:, :, 1, :25, :]], axis=2)
    y1p = jnp.pad(y1, ((0, 0), (1, 1), (1, 1), (0, 0)))
    yr = y1p.reshape(16, 26, 2, 26, 2, B).transpose(2, 4, 0, 1, 3, 5)
    yps = jnp.pad(yr, ((0, 0),) * 4 + ((0, 14), (0, 0)))

    w2 = conv2_w.transpose(0, 2, 3, 1).reshape(8, 144).astype(BF)
    b2 = conv2_b.reshape(8, 1).astype(F32)
    t = jnp.pad(fc1_w.reshape(8, 25, 25, 64), ((0, 0), (0, 0), (0, 7), (0, 0)))
    fc1p = t.transpose(3, 0, 1, 2).reshape(64, 6400).astype(BF)
    b3 = fc1_b.reshape(64, 1).astype(F32)
    w4 = fc2_w.T.astype(BF)
    b4 = fc2_b.reshape(2, 1).astype(F32)

    out = pl.pallas_call(
        _conv2_mlp_kernel,
        out_shape=jax.ShapeDtypeStruct((2, B), F32),
        grid=(G,),
        in_specs=[
            pl.BlockSpec((2, 2, 16, 26, 40, 128),
                         lambda i: (0, 0, 0, 0, 0, i)),
            pl.BlockSpec((8, 144), lambda i: (0, 0)),
            pl.BlockSpec((8, 1), lambda i: (0, 0)),
            pl.BlockSpec((64, 6400), lambda i: (0, 0)),
            pl.BlockSpec((64, 1), lambda i: (0, 0)),
            pl.BlockSpec((2, 64), lambda i: (0, 0)),
            pl.BlockSpec((2, 1), lambda i: (0, 0)),
        ],
        out_specs=pl.BlockSpec((2, 128), lambda i: (0, i)),
        scratch_shapes=[pltpu.VMEM((144, 5, 32, 128), BF),
                        pltpu.VMEM((8, 25, 32, 128), BF)],
        compiler_params=pltpu.CompilerParams(
            dimension_semantics=("parallel",),
            vmem_limit_bytes=48 * 1024 * 1024),
    )(yps, w2, b2, fc1p, b3, w4, b4)
    return out.T
